# baseline (speedup 1.0000x reference)
import functools

import jax
import jax.numpy as jnp
from jax import lax
from jax.experimental import pallas as pl
from jax.experimental.pallas import tpu as pltpu
from jax.experimental.pallas import tpu_sc as plsc

NUM_CORES = 2
NUM_SUBCORES = 16
NUM_WORKERS = NUM_CORES * NUM_SUBCORES
ROWS_PER_CHUNK = 4
NBUF = 2


@functools.lru_cache(maxsize=None)
def _make_gather(B0, H, V, D):
    rows_per_w = B0 // NUM_WORKERS
    n_chunks = rows_per_w // ROWS_PER_CHUNK
    assert n_chunks % NBUF == 0
    mesh = plsc.VectorSubcoreMesh(
        core_axis_name="c",
        subcore_axis_name="s",
        num_cores=NUM_CORES,
        num_subcores=NUM_SUBCORES,
    )

    @functools.partial(
        pl.kernel,
        out_type=jax.ShapeDtypeStruct((B0, H, D), jnp.float32),
        mesh=mesh,
        scratch_types=[
            pltpu.VMEM((NBUF, ROWS_PER_CHUNK, H), jnp.int32),
            pltpu.VMEM((NBUF, ROWS_PER_CHUNK, H, D), jnp.float32),
        ]
        + [pltpu.SemaphoreType.DMA] * (3 * NBUF),
        compiler_params=pltpu.CompilerParams(use_tc_tiling_on_sc=False),
    )
    def gather_kernel(idx_hbm, table_hbm, out_hbm, idx_v, rows_v, *sems):
        sem_i = sems[0:NBUF]
        sem_g = sems[NBUF : 2 * NBUF]
        sem_o = sems[2 * NBUF : 3 * NBUF]
        wid = lax.axis_index("s") * NUM_CORES + lax.axis_index("c")
        base = wid * rows_per_w

        for b in range(NBUF):
            off = base + b * ROWS_PER_CHUNK
            pltpu.async_copy(
                idx_hbm.at[pl.ds(off, ROWS_PER_CHUNK)], idx_v.at[b], sem_i[b]
            )

        def outer_body(o, carry):
            for b in range(NBUF):
                c = o * NBUF + b
                off = base + c * ROWS_PER_CHUNK
                pltpu.make_async_copy(
                    idx_hbm.at[pl.ds(off, ROWS_PER_CHUNK)], idx_v.at[b], sem_i[b]
                ).wait()

                @pl.when(c >= NBUF)
                def _():
                    poff = base + (c - NBUF) * ROWS_PER_CHUNK
                    pltpu.make_async_copy(
                        rows_v.at[b],
                        out_hbm.at[pl.ds(poff, ROWS_PER_CHUNK)],
                        sem_o[b],
                    ).wait()

                for r in range(ROWS_PER_CHUNK):
                    pltpu.async_copy(
                        table_hbm.at[idx_v.at[b, r]], rows_v.at[b, r], sem_g[b]
                    )
                for r in range(ROWS_PER_CHUNK):
                    pltpu.make_async_copy(
                        table_hbm.at[idx_v.at[b, r]], rows_v.at[b, r], sem_g[b]
                    ).wait()

                pltpu.async_copy(
                    rows_v.at[b], out_hbm.at[pl.ds(off, ROWS_PER_CHUNK)], sem_o[b]
                )

                @pl.when(c + NBUF < n_chunks)
                def _():
                    noff = base + (c + NBUF) * ROWS_PER_CHUNK
                    pltpu.async_copy(
                        idx_hbm.at[pl.ds(noff, ROWS_PER_CHUNK)], idx_v.at[b], sem_i[b]
                    )

            return carry

        lax.fori_loop(0, n_chunks // NBUF, outer_body, 0)

        for b in range(NBUF):
            c = n_chunks - NBUF + b
            off = base + c * ROWS_PER_CHUNK
            pltpu.make_async_copy(
                rows_v.at[b], out_hbm.at[pl.ds(off, ROWS_PER_CHUNK)], sem_o[b]
            ).wait()

    return gather_kernel


def kernel(indices, table):
    B0, H = indices.shape
    V, D = table.shape
    return _make_gather(B0, H, V, D)(indices.astype(jnp.int32), table)


# speedup vs baseline: 1.0827x; 1.0827x over previous
import functools

import jax
import jax.numpy as jnp
from jax import lax
from jax.experimental import pallas as pl
from jax.experimental.pallas import tpu as pltpu
from jax.experimental.pallas import tpu_sc as plsc

NUM_CORES = 2
NUM_SUBCORES = 16
NUM_WORKERS = NUM_CORES * NUM_SUBCORES
CHUNK_B = 8
DP = 128


@functools.lru_cache(maxsize=None)
def _make_gather(B0, H, V, D):
    rows_per_w = B0 // NUM_WORKERS
    n_chunks = rows_per_w // CHUNK_B
    assert n_chunks % 2 == 0
    mesh = plsc.VectorSubcoreMesh(
        core_axis_name="c",
        subcore_axis_name="s",
        num_cores=NUM_CORES,
        num_subcores=NUM_SUBCORES,
    )

    @functools.partial(
        pl.kernel,
        out_type=jax.ShapeDtypeStruct((B0, H, D), jnp.float32),
        mesh=mesh,
        scratch_types=[
            pltpu.VMEM((CHUNK_B * H,), jnp.int32),
            pltpu.VMEM((CHUNK_B * H,), jnp.int32),
            pltpu.VMEM((H, DP), jnp.float32),
            pltpu.VMEM((H, DP), jnp.float32),
            pltpu.VMEM((H, D), jnp.float32),
            pltpu.VMEM((H, D), jnp.float32),
        ]
        + [pltpu.SemaphoreType.DMA] * 6,
    )
    def gather_kernel(
        idx_hbm, table_hbm, out_hbm, idx0, idx1, pad0, pad1, row0, row1, *sems
    ):
        idx_v = (idx0, idx1)
        pad_v = (pad0, pad1)
        row_v = (row0, row1)
        sem_i = sems[0:2]
        sem_g = sems[2:4]
        sem_o = sems[4:6]
        wid = lax.axis_index("s") * NUM_CORES + lax.axis_index("c")
        base = wid * rows_per_w

        def idx_copy(c, ci):
            return pltpu.async_copy(
                idx_hbm.at[pl.ds((base + c * CHUNK_B) * H, CHUNK_B * H)],
                idx_v[ci],
                sem_i[ci],
            )

        def idx_wait(c, ci):
            pltpu.make_async_copy(
                idx_hbm.at[pl.ds((base + c * CHUNK_B) * H, CHUNK_B * H)],
                idx_v[ci],
                sem_i[ci],
            ).wait()

        def gather(ci, r, gb):
            return pltpu.async_copy(
                table_hbm.at[idx_v[ci].at[pl.ds(H * r, H)]], pad_v[gb], sem_g[gb]
            )

        def gather_wait(ci, r, gb):
            pltpu.make_async_copy(
                table_hbm.at[idx_v[ci].at[pl.ds(H * r, H)]], pad_v[gb], sem_g[gb]
            ).wait()

        def wb(j, ob):
            return pltpu.async_copy(row_v[ob], out_hbm.at[base + j], sem_o[ob])

        def wb_wait(j, ob):
            pltpu.make_async_copy(row_v[ob], out_hbm.at[base + j], sem_o[ob]).wait()

        def strip(gb, ob):
            def body(r2, carry):
                for rr in range(2):
                    r = r2 * 2 + rr
                    for kk in range(D // 16):
                        row_v[ob][r, pl.ds(16 * kk, 16)] = pad_v[gb][
                            r, pl.ds(16 * kk, 16)
                        ]
                return carry

            lax.fori_loop(0, H // 2, body, 0)

        idx_copy(0, 0)
        idx_wait(0, 0)
        gather(0, 0, 0)

        def outer_body(o, carry):
            for cc in range(2):
                c = o * 2 + cc
                for r in range(CHUNK_B):
                    j = c * CHUNK_B + r
                    gb = r % 2
                    ob = r % 2
                    ci = cc
                    if r == 0:
                        if cc == 0:
                            idx_copy(c + 1, 1)
                        else:
                            def prefetch():
                                idx_copy(c + 1, 0)
                                return None

                            pl.when(o < n_chunks // 2 - 1)(prefetch)
                    gather_wait(ci, r, gb)
                    if r == CHUNK_B - 1:
                        nci = (cc + 1) % 2

                        def fire_next():
                            idx_wait(c + 1, nci)
                            gather(nci, 0, (gb + 1) % 2)
                            return None

                        if cc == 0:
                            fire_next()
                        else:
                            pl.when(o < n_chunks // 2 - 1)(fire_next)
                    else:
                        gather(ci, r + 1, (gb + 1) % 2)
                    def rowbuf_free():
                        wb_wait(j - 2, ob)

                    if r >= 2 or cc == 1:
                        rowbuf_free()
                    else:
                        pl.when(o >= 1)(rowbuf_free)
                    strip(gb, ob)
                    wb(j, ob)
            return carry

        lax.fori_loop(0, n_chunks // 2, outer_body, 0)

        j_last = rows_per_w - 1
        wb_wait(j_last - 1, (j_last - 1) % 2)
        wb_wait(j_last, j_last % 2)

    return gather_kernel


def kernel(indices, table):
    B0, H = indices.shape
    V, D = table.shape
    table_wide = jnp.pad(table, ((0, 0), (0, DP - D)))
    idx_flat = indices.reshape(B0 * H).astype(jnp.int32)
    return _make_gather(B0, H, V, D)(idx_flat, table_wide)
